# Initial kernel scaffold; baseline (speedup 1.0000x reference)
import jax
import jax.numpy as jnp
from jax.experimental import pallas as pl


def kernel(initial_spikes, input_frames, stacked_flat_spat_filters, stacked_feedback_filters, stacked_coupling_filters, stacked_bias, coupled_sel, forward_sel, forward_weights):
    raise NotImplementedError("write your pallas kernel here")


# speedup vs baseline: 10.0190x; 10.0190x over previous
import functools

import jax
import jax.numpy as jnp
from jax import lax
from jax.experimental import pallas as pl
from jax.experimental.pallas import tpu as pltpu

N_CELLS = 512
N_PIXELS = 16384
N_TAPS = 100
MAX_COUPLED = 32
N_FRAMES = 60
N_BINS = 300
N_INIT = 100

N_SLOTS = MAX_COUPLED + 1
N_EDGES = N_SLOTS * N_CELLS
PIX_CHUNK = 2048
N_PIX_CHUNKS = N_PIXELS // PIX_CHUNK
M_PAD = 64


def _stim_body(frames_ref, filt_ref, fsel_ref, fw_ref, bias_ref,
               out_ref, acc_ref):
    k = pl.program_id(0)

    @pl.when(k == 0)
    def _init():
        acc_ref[...] = jnp.zeros_like(acc_ref)

    acc_ref[...] += lax.dot_general(
        frames_ref[...], filt_ref[...],
        dimension_numbers=(((1,), (1,)), ((), ())),
        preferred_element_type=jnp.float32)

    @pl.when(k == N_PIX_CHUNKS - 1)
    def _finish():
        spat = acc_ref[0:N_FRAMES, :]
        frame_ids = lax.broadcasted_iota(jnp.int32, (N_BINS, N_FRAMES), 1)
        u = (fw_ref[:, 0:1] * (fsel_ref[:, 0:1] == frame_ids) +
             fw_ref[:, 1:2] * (fsel_ref[:, 1:2] == frame_ids)).astype(jnp.float32)
        out_ref[...] = lax.dot_general(
            u, spat,
            dimension_numbers=(((1,), (0,)), ((), ())),
            preferred_element_type=jnp.float32) + bias_ref[...]


ACC_ROWS = 408
SHIFT_W = 112


def _sim_body(stim_ref, init_ref, frev_ref, e_ref, out_ref, acc_ref):
    acc_ref[...] = jnp.zeros_like(acc_ref)
    out_ref[0:N_INIT, :] = init_ref[0:N_INIT, :]
    iota8 = lax.broadcasted_iota(jnp.int32, (8, 1), 0)
    sh_u = lax.broadcasted_iota(jnp.int32, (SHIFT_W, SHIFT_W), 0)
    sh_v = lax.broadcasted_iota(jnp.int32, (SHIFT_W, SHIFT_W), 1)
    zeros12 = jnp.zeros((SHIFT_W - N_TAPS, N_CELLS), jnp.float32)

    def _row8(ref, base, d):
        blk = ref[pl.ds(pl.multiple_of(base, 8), 8), :]
        return jnp.sum(jnp.where(iota8 == d, blk, 0.0), axis=0, keepdims=True)

    def step(t, _):
        base = pl.multiple_of((t // 8) * 8, 8)
        d = t - base
        accrow = _row8(acc_ref, base, d)
        tm1 = jnp.maximum(t - 1, 0)
        stimrow = _row8(stim_ref, pl.multiple_of((tm1 // 8) * 8, 8),
                        tm1 - (tm1 // 8) * 8)
        s_gen = jax.nn.sigmoid(stimrow + accrow)
        s_init = _row8(init_ref, jnp.minimum(base, N_INIT - 4), d)
        s = jnp.where(t >= N_INIT, s_gen, s_init)

        blk = out_ref[pl.ds(base, 8), :]
        out_ref[pl.ds(base, 8), :] = jnp.where(
            (iota8 == d) & (t >= N_INIT), s, blk)

        routed = lax.dot_general(
            s, e_ref[...],
            dimension_numbers=(((1,), (0,)), ((), ())),
            preferred_element_type=jnp.float32)
        contrib = jnp.sum(
            (frev_ref[...] * routed).reshape(N_TAPS, N_SLOTS, N_CELLS),
            axis=1)
        cz = jnp.concatenate([contrib, zeros12], axis=0)

        wbase = pl.multiple_of(((t + 1) // 8) * 8, 8)
        d1 = (t + 1) - ((t + 1) // 8) * 8
        shmat = (sh_v == sh_u - d1).astype(jnp.float32)
        shifted = lax.dot_general(
            shmat, cz,
            dimension_numbers=(((1,), (0,)), ((), ())),
            preferred_element_type=jnp.float32)
        acc_ref[pl.ds(wbase, SHIFT_W), :] += shifted
        return 0

    lax.fori_loop(0, N_BINS, step, 0)


@jax.jit
def kernel(initial_spikes, input_frames, stacked_flat_spat_filters,
           stacked_feedback_filters, stacked_coupling_filters, stacked_bias,
           coupled_sel, forward_sel, forward_weights):
    frames_p = jnp.pad(input_frames, ((0, M_PAD - N_FRAMES), (0, 0)))

    stim_t = pl.pallas_call(
        _stim_body,
        grid=(N_PIX_CHUNKS,),
        in_specs=[
            pl.BlockSpec((M_PAD, PIX_CHUNK), lambda k: (0, k)),
            pl.BlockSpec((N_CELLS, PIX_CHUNK), lambda k: (0, k)),
            pl.BlockSpec((N_BINS, 2), lambda k: (0, 0)),
            pl.BlockSpec((N_BINS, 2), lambda k: (0, 0)),
            pl.BlockSpec((1, N_CELLS), lambda k: (0, 0)),
        ],
        out_specs=pl.BlockSpec((N_BINS, N_CELLS), lambda k: (0, 0)),
        out_shape=jax.ShapeDtypeStruct((N_BINS, N_CELLS), jnp.float32),
        scratch_shapes=[pltpu.VMEM((M_PAD, N_CELLS), jnp.float32)],
    )(frames_p, stacked_flat_spat_filters,
      forward_sel.astype(jnp.int32), forward_weights,
      stacked_bias.reshape(1, N_CELLS))

    sel_ext = jnp.concatenate(
        [coupled_sel.astype(jnp.int32).T,
         jnp.arange(N_CELLS, dtype=jnp.int32)[None, :]], axis=0)
    e_mat = (jnp.arange(N_CELLS, dtype=jnp.int32)[:, None]
             == sel_ext.reshape(1, N_EDGES)).astype(jnp.float32)
    f_ext = jnp.concatenate(
        [stacked_coupling_filters.transpose(2, 1, 0),
         stacked_feedback_filters.T[:, None, :]], axis=1)
    f_rev = f_ext.reshape(N_TAPS, N_EDGES)[::-1]

    stim_p = jnp.pad(stim_t, ((0, 4), (0, 0)))
    init_p = jnp.pad(initial_spikes.T, ((0, 4), (0, 0)))

    out_t = pl.pallas_call(
        _sim_body,
        out_shape=jax.ShapeDtypeStruct((N_BINS + 4, N_CELLS), jnp.float32),
        scratch_shapes=[pltpu.VMEM((ACC_ROWS, N_CELLS), jnp.float32)],
    )(stim_p, init_p, f_rev, e_mat)

    return out_t[:N_BINS].T


# speedup vs baseline: 15.0470x; 1.5018x over previous
import functools

import jax
import jax.numpy as jnp
from jax import lax
from jax.experimental import pallas as pl
from jax.experimental.pallas import tpu as pltpu

N_CELLS = 512
N_PIXELS = 16384
N_TAPS = 100
MAX_COUPLED = 32
N_FRAMES = 60
N_BINS = 300
N_INIT = 100

N_SLOTS = MAX_COUPLED + 1
N_EDGES = N_SLOTS * N_CELLS
PIX_CHUNK = 2048
N_PIX_CHUNKS = N_PIXELS // PIX_CHUNK
M_PAD = 64


def _stim_body(frames_ref, filt_ref, fsel_ref, fw_ref, bias_ref,
               out_ref, acc_ref):
    k = pl.program_id(0)

    @pl.when(k == 0)
    def _init():
        acc_ref[...] = jnp.zeros_like(acc_ref)

    acc_ref[...] += lax.dot_general(
        frames_ref[...], filt_ref[...],
        dimension_numbers=(((1,), (1,)), ((), ())),
        preferred_element_type=jnp.float32)

    @pl.when(k == N_PIX_CHUNKS - 1)
    def _finish():
        spat = acc_ref[0:N_FRAMES, :]
        frame_ids = lax.broadcasted_iota(jnp.int32, (N_BINS, N_FRAMES), 1)
        u = (fw_ref[:, 0:1] * (fsel_ref[:, 0:1] == frame_ids) +
             fw_ref[:, 1:2] * (fsel_ref[:, 1:2] == frame_ids)).astype(jnp.float32)
        out_ref[...] = lax.dot_general(
            u, spat,
            dimension_numbers=(((1,), (0,)), ((), ())),
            preferred_element_type=jnp.float32) + bias_ref[...]


ACC_ROWS = 408
SHIFT_W = 112


def _sim_body(stim_ref, init_ref, frev_ref, e_ref, out_ref, acc_ref):
    acc_ref[...] = jnp.zeros_like(acc_ref)
    out_ref[0:N_INIT, :] = init_ref[0:N_INIT, :]
    iota8 = lax.broadcasted_iota(jnp.int32, (8, 1), 0)
    sh_u = lax.broadcasted_iota(jnp.int32, (SHIFT_W, N_TAPS), 0)
    sh_v = lax.broadcasted_iota(jnp.int32, (SHIFT_W, N_TAPS), 1)

    def _row8(ref, base, d):
        blk = ref[pl.ds(pl.multiple_of(base, 8), 8), :]
        return jnp.sum(jnp.where(iota8 == d, blk, 0.0), axis=0, keepdims=True)

    def step(t, _):
        base = pl.multiple_of((t // 8) * 8, 8)
        d = t - base
        accrow = _row8(acc_ref, base, d)
        tm1 = jnp.maximum(t - 1, 0)
        stimrow = _row8(stim_ref, pl.multiple_of((tm1 // 8) * 8, 8),
                        tm1 - (tm1 // 8) * 8)
        s_gen = jax.nn.sigmoid(stimrow + accrow)
        s_init = _row8(init_ref, jnp.minimum(base, N_INIT - 4), d)
        s = jnp.where(t >= N_INIT, s_gen, s_init)

        blk = out_ref[pl.ds(base, 8), :]
        out_ref[pl.ds(base, 8), :] = jnp.where(
            (iota8 == d) & (t >= N_INIT), s, blk)

        routed = lax.dot_general(
            s.astype(jnp.bfloat16), e_ref[...],
            dimension_numbers=(((1,), (0,)), ((), ())),
            preferred_element_type=jnp.float32)
        contrib = jnp.zeros((N_TAPS, N_CELLS), jnp.float32)
        for k in range(N_SLOTS):
            sl = slice(k * N_CELLS, (k + 1) * N_CELLS)
            contrib = contrib + frev_ref[:, sl] * routed[:, sl]

        wbase = pl.multiple_of(((t + 1) // 8) * 8, 8)
        d1 = (t + 1) - ((t + 1) // 8) * 8
        shmat = (sh_v == sh_u - d1).astype(jnp.float32)
        shifted = lax.dot_general(
            shmat, contrib,
            dimension_numbers=(((1,), (0,)), ((), ())),
            preferred_element_type=jnp.float32)
        acc_ref[pl.ds(wbase, SHIFT_W), :] += shifted
        return 0

    lax.fori_loop(0, N_BINS, step, 0)


@jax.jit
def kernel(initial_spikes, input_frames, stacked_flat_spat_filters,
           stacked_feedback_filters, stacked_coupling_filters, stacked_bias,
           coupled_sel, forward_sel, forward_weights):
    frames_p = jnp.pad(input_frames, ((0, M_PAD - N_FRAMES), (0, 0)))

    stim_t = pl.pallas_call(
        _stim_body,
        grid=(N_PIX_CHUNKS,),
        in_specs=[
            pl.BlockSpec((M_PAD, PIX_CHUNK), lambda k: (0, k)),
            pl.BlockSpec((N_CELLS, PIX_CHUNK), lambda k: (0, k)),
            pl.BlockSpec((N_BINS, 2), lambda k: (0, 0)),
            pl.BlockSpec((N_BINS, 2), lambda k: (0, 0)),
            pl.BlockSpec((1, N_CELLS), lambda k: (0, 0)),
        ],
        out_specs=pl.BlockSpec((N_BINS, N_CELLS), lambda k: (0, 0)),
        out_shape=jax.ShapeDtypeStruct((N_BINS, N_CELLS), jnp.float32),
        scratch_shapes=[pltpu.VMEM((M_PAD, N_CELLS), jnp.float32)],
    )(frames_p, stacked_flat_spat_filters,
      forward_sel.astype(jnp.int32), forward_weights,
      stacked_bias.reshape(1, N_CELLS))

    sel_ext = jnp.concatenate(
        [coupled_sel.astype(jnp.int32).T,
         jnp.arange(N_CELLS, dtype=jnp.int32)[None, :]], axis=0)
    e_mat = (jnp.arange(N_CELLS, dtype=jnp.int32)[:, None]
             == sel_ext.reshape(1, N_EDGES)).astype(jnp.bfloat16)
    f_ext = jnp.concatenate(
        [stacked_coupling_filters.transpose(2, 1, 0),
         stacked_feedback_filters.T[:, None, :]], axis=1)
    f_rev = f_ext.reshape(N_TAPS, N_EDGES)[::-1]

    stim_p = jnp.pad(stim_t, ((0, 4), (0, 0)))
    init_p = jnp.pad(initial_spikes.T, ((0, 4), (0, 0)))

    out_t = pl.pallas_call(
        _sim_body,
        out_shape=jax.ShapeDtypeStruct((N_BINS + 4, N_CELLS), jnp.float32),
        scratch_shapes=[pltpu.VMEM((ACC_ROWS, N_CELLS), jnp.float32)],
    )(stim_p, init_p, f_rev, e_mat)

    return out_t[:N_BINS].T


# speedup vs baseline: 18.6943x; 1.2424x over previous
import functools

import jax
import jax.numpy as jnp
from jax import lax
from jax.experimental import pallas as pl
from jax.experimental.pallas import tpu as pltpu

N_CELLS = 512
N_PIXELS = 16384
N_TAPS = 100
MAX_COUPLED = 32
N_FRAMES = 60
N_BINS = 300
N_INIT = 100

N_SLOTS = MAX_COUPLED + 1
N_EDGES = N_SLOTS * N_CELLS
PIX_CHUNK = 2048
N_PIX_CHUNKS = N_PIXELS // PIX_CHUNK
M_PAD = 64


def _stim_body(frames_ref, filt_ref, fsel_ref, fw_ref, bias_ref,
               out_ref, acc_ref):
    k = pl.program_id(0)

    @pl.when(k == 0)
    def _init():
        acc_ref[...] = jnp.zeros_like(acc_ref)

    acc_ref[...] += lax.dot_general(
        frames_ref[...], filt_ref[...],
        dimension_numbers=(((1,), (1,)), ((), ())),
        preferred_element_type=jnp.float32)

    @pl.when(k == N_PIX_CHUNKS - 1)
    def _finish():
        spat = acc_ref[0:N_FRAMES, :]
        frame_ids = lax.broadcasted_iota(jnp.int32, (N_BINS, N_FRAMES), 1)
        u = (fw_ref[:, 0:1] * (fsel_ref[:, 0:1] == frame_ids) +
             fw_ref[:, 1:2] * (fsel_ref[:, 1:2] == frame_ids)).astype(jnp.float32)
        out_ref[...] = lax.dot_general(
            u, spat,
            dimension_numbers=(((1,), (0,)), ((), ())),
            preferred_element_type=jnp.float32) + bias_ref[...]


ACC_ROWS = 408
SHIFT_W = 112


def _fold(frev_ref, routed):
    contrib = jnp.zeros((N_TAPS, N_CELLS), jnp.float32)
    for k in range(N_SLOTS):
        sl = slice(k * N_CELLS, (k + 1) * N_CELLS)
        contrib = contrib + frev_ref[:, sl] * routed[:, sl]
    return contrib


def _sim_body(stim_ref, init_ref, frev_ref, e_ref, out_ref, acc_ref, ri_ref):
    acc_ref[...] = jnp.zeros_like(acc_ref)
    out_ref[0:N_INIT, :] = init_ref[0:N_INIT, :]
    iota8 = lax.broadcasted_iota(jnp.int32, (8, 1), 0)
    sh_u = lax.broadcasted_iota(jnp.int32, (SHIFT_W, N_TAPS), 0)
    sh_v = lax.broadcasted_iota(jnp.int32, (SHIFT_W, N_TAPS), 1)

    ri_ref[...] = lax.dot_general(
        init_ref[...].astype(jnp.bfloat16), e_ref[...],
        dimension_numbers=(((1,), (0,)), ((), ())),
        preferred_element_type=jnp.float32)

    def _row8(ref, base, d):
        blk = ref[pl.ds(pl.multiple_of(base, 8), 8), :]
        return jnp.sum(jnp.where(iota8 == d, blk, 0.0), axis=0, keepdims=True)

    def _scatter(t, contrib):
        wbase = pl.multiple_of(((t + 1) // 8) * 8, 8)
        d1 = (t + 1) - ((t + 1) // 8) * 8
        shmat = (sh_v == sh_u - d1).astype(jnp.float32)
        shifted = lax.dot_general(
            shmat, contrib,
            dimension_numbers=(((1,), (0,)), ((), ())),
            preferred_element_type=jnp.float32)
        acc_ref[pl.ds(wbase, SHIFT_W), :] += shifted

    def init_step(t, _):
        routed = _row8(ri_ref, (t // 8) * 8, t - (t // 8) * 8)
        _scatter(t, _fold(frev_ref, routed))
        return 0

    lax.fori_loop(0, N_INIT, init_step, 0)

    def gen_step(t, _):
        accrow = _row8(acc_ref, (t // 8) * 8, t - (t // 8) * 8)
        tm1 = t - 1
        stimrow = _row8(stim_ref, (tm1 // 8) * 8, tm1 - (tm1 // 8) * 8)
        s = jax.nn.sigmoid(stimrow + accrow)

        base = pl.multiple_of((t // 8) * 8, 8)
        blk = out_ref[pl.ds(base, 8), :]
        out_ref[pl.ds(base, 8), :] = jnp.where(iota8 == t - base, s, blk)

        routed = lax.dot_general(
            s.astype(jnp.bfloat16), e_ref[...],
            dimension_numbers=(((1,), (0,)), ((), ())),
            preferred_element_type=jnp.float32)
        _scatter(t, _fold(frev_ref, routed))
        return 0

    lax.fori_loop(N_INIT, N_BINS, gen_step, 0)


@jax.jit
def kernel(initial_spikes, input_frames, stacked_flat_spat_filters,
           stacked_feedback_filters, stacked_coupling_filters, stacked_bias,
           coupled_sel, forward_sel, forward_weights):
    frames_p = jnp.pad(input_frames, ((0, M_PAD - N_FRAMES), (0, 0)))

    stim_t = pl.pallas_call(
        _stim_body,
        grid=(N_PIX_CHUNKS,),
        in_specs=[
            pl.BlockSpec((M_PAD, PIX_CHUNK), lambda k: (0, k)),
            pl.BlockSpec((N_CELLS, PIX_CHUNK), lambda k: (0, k)),
            pl.BlockSpec((N_BINS, 2), lambda k: (0, 0)),
            pl.BlockSpec((N_BINS, 2), lambda k: (0, 0)),
            pl.BlockSpec((1, N_CELLS), lambda k: (0, 0)),
        ],
        out_specs=pl.BlockSpec((N_BINS, N_CELLS), lambda k: (0, 0)),
        out_shape=jax.ShapeDtypeStruct((N_BINS, N_CELLS), jnp.float32),
        scratch_shapes=[pltpu.VMEM((M_PAD, N_CELLS), jnp.float32)],
    )(frames_p, stacked_flat_spat_filters,
      forward_sel.astype(jnp.int32), forward_weights,
      stacked_bias.reshape(1, N_CELLS))

    sel_ext = jnp.concatenate(
        [coupled_sel.astype(jnp.int32).T,
         jnp.arange(N_CELLS, dtype=jnp.int32)[None, :]], axis=0)
    e_mat = (jnp.arange(N_CELLS, dtype=jnp.int32)[:, None]
             == sel_ext.reshape(1, N_EDGES)).astype(jnp.bfloat16)
    f_ext = jnp.concatenate(
        [stacked_coupling_filters.transpose(2, 1, 0),
         stacked_feedback_filters.T[:, None, :]], axis=1)
    f_rev = f_ext.reshape(N_TAPS, N_EDGES)[::-1]

    stim_p = jnp.pad(stim_t, ((0, 4), (0, 0)))
    init_p = jnp.pad(initial_spikes.T, ((0, 4), (0, 0)))

    out_t = pl.pallas_call(
        _sim_body,
        out_shape=jax.ShapeDtypeStruct((N_BINS + 4, N_CELLS), jnp.float32),
        scratch_shapes=[pltpu.VMEM((ACC_ROWS, N_CELLS), jnp.float32),
                        pltpu.VMEM((N_INIT + 4, N_EDGES), jnp.float32)],
    )(stim_p, init_p, f_rev, e_mat)

    return out_t[:N_BINS].T


# speedup vs baseline: 23.9787x; 1.2827x over previous
import functools

import jax
import jax.numpy as jnp
from jax import lax
from jax.experimental import pallas as pl
from jax.experimental.pallas import tpu as pltpu

N_CELLS = 512
N_PIXELS = 16384
N_TAPS = 100
MAX_COUPLED = 32
N_FRAMES = 60
N_BINS = 300
N_INIT = 100

N_SLOTS = MAX_COUPLED + 1
N_EDGES = N_SLOTS * N_CELLS
PIX_CHUNK = 2048
N_PIX_CHUNKS = N_PIXELS // PIX_CHUNK
M_PAD = 64


def _stim_body(frames_ref, filt_ref, fsel_ref, fw_ref, bias_ref,
               out_ref, acc_ref):
    k = pl.program_id(0)

    @pl.when(k == 0)
    def _init():
        acc_ref[...] = jnp.zeros_like(acc_ref)

    acc_ref[...] += lax.dot_general(
        frames_ref[...], filt_ref[...],
        dimension_numbers=(((1,), (1,)), ((), ())),
        preferred_element_type=jnp.float32)

    @pl.when(k == N_PIX_CHUNKS - 1)
    def _finish():
        spat = acc_ref[0:N_FRAMES, :]
        frame_ids = lax.broadcasted_iota(jnp.int32, (N_BINS, N_FRAMES), 1)
        u = (fw_ref[:, 0:1] * (fsel_ref[:, 0:1] == frame_ids) +
             fw_ref[:, 1:2] * (fsel_ref[:, 1:2] == frame_ids)).astype(jnp.float32)
        out_ref[...] = lax.dot_general(
            u, spat,
            dimension_numbers=(((1,), (0,)), ((), ())),
            preferred_element_type=jnp.float32) + bias_ref[...]


ACC_ROWS = 416
N_RECENT = 7
SHIFT_W = 112


def _fold(frev_ref, routed):
    contrib = jnp.zeros((N_TAPS, N_CELLS), jnp.float32)
    for k in range(N_SLOTS):
        sl = slice(k * N_CELLS, (k + 1) * N_CELLS)
        contrib = contrib + frev_ref[:, sl] * routed[:, sl]
    return contrib


def _sim_body(stim_ref, init_ref, frev_ref, e_ref, m_ref, out_ref, acc_ref,
              ri_ref):
    acc_ref[...] = jnp.zeros_like(acc_ref)
    out_ref[0:N_INIT, :] = init_ref[0:N_INIT, :]
    iota8 = lax.broadcasted_iota(jnp.int32, (8, 1), 0)
    sh_u = lax.broadcasted_iota(jnp.int32, (SHIFT_W, N_TAPS), 0)
    sh_v = lax.broadcasted_iota(jnp.int32, (SHIFT_W, N_TAPS), 1)

    ri_ref[...] = lax.dot_general(
        init_ref[...].astype(jnp.bfloat16), e_ref[...],
        dimension_numbers=(((1,), (0,)), ((), ())),
        preferred_element_type=jnp.float32)

    def _row8(ref, base, d):
        blk = ref[pl.ds(pl.multiple_of(base, 8), 8), :]
        return jnp.sum(jnp.where(iota8 == d, blk, 0.0), axis=0, keepdims=True)

    def _scatter(t, contrib):
        wbase = pl.multiple_of(((t + 1) // 8) * 8, 8)
        d1 = (t + 1) - ((t + 1) // 8) * 8
        shmat = (sh_v == sh_u - d1).astype(jnp.float32)
        shifted = lax.dot_general(
            shmat, contrib,
            dimension_numbers=(((1,), (0,)), ((), ())),
            preferred_element_type=jnp.float32)
        acc_ref[pl.ds(wbase, SHIFT_W), :] += shifted

    rowiota = lax.broadcasted_iota(jnp.int32, (N_TAPS, 1), 0)

    def init_step(t, _):
        routed = _row8(ri_ref, (t // 8) * 8, t - (t // 8) * 8)
        contrib = _fold(frev_ref, routed)
        keep = jnp.logical_not((rowiota < N_RECENT)
                               & (t + 1 + rowiota >= N_INIT))
        _scatter(t, jnp.where(keep, contrib, 0.0))
        return 0

    lax.fori_loop(0, N_INIT, init_step, 0)

    sprev0 = jnp.concatenate(
        [init_ref[N_INIT - 1 - j:N_INIT - j, :] for j in range(N_RECENT)],
        axis=0)

    def gen_group(g, sprev):
        s_list = []
        for b in range(8):
            tbase = pl.multiple_of(g * 8 + (96 if b < 4 else 104), 8)
            trow = (4 + b) % 8
            accrow = acc_ref[pl.ds(tbase, 8), :][trow:trow + 1, :]
            sbase = pl.multiple_of(g * 8 + (96 if b < 5 else 104), 8)
            srow = (3 + b) % 8
            stimrow = stim_ref[pl.ds(sbase, 8), :][srow:srow + 1, :]
            corr = lax.dot_general(
                sprev.reshape(1, N_RECENT * N_CELLS).astype(jnp.bfloat16),
                m_ref[...],
                dimension_numbers=(((1,), (0,)), ((), ())),
                preferred_element_type=jnp.float32)
            s = jax.nn.sigmoid(stimrow + accrow + corr)

            blk = out_ref[pl.ds(tbase, 8), :]
            out_ref[pl.ds(tbase, 8), :] = jnp.where(iota8 == trow, s, blk)

            s_list.append(s)
            sprev = jnp.concatenate([s, sprev[0:N_RECENT - 1, :]], axis=0)

        s8 = jnp.concatenate(s_list, axis=0)
        routed8 = lax.dot_general(
            s8.astype(jnp.bfloat16), e_ref[...],
            dimension_numbers=(((1,), (0,)), ((), ())),
            preferred_element_type=jnp.float32)
        c8 = jnp.zeros((8, N_TAPS, N_CELLS), jnp.float32)
        for k in range(N_SLOTS):
            sl = slice(k * N_CELLS, (k + 1) * N_CELLS)
            c8 = c8 + frev_ref[:, sl][None, :, :] * routed8[:, sl][:, None, :]

        for b in range(8):
            d1 = (4 + b) % 8
            wbase = pl.multiple_of(g * 8 + (104 if b < 4 else 112), 8)
            clate = c8[b, N_RECENT:, :]
            parts = []
            if d1 > 0:
                parts.append(jnp.zeros((d1, N_CELLS), jnp.float32))
            parts.append(clate)
            parts.append(jnp.zeros((SHIFT_W - (N_TAPS - N_RECENT) - d1,
                                    N_CELLS), jnp.float32))
            acc_ref[pl.ds(wbase, SHIFT_W), :] += jnp.concatenate(parts,
                                                                 axis=0)
        return sprev

    lax.fori_loop(0, (N_BINS - N_INIT) // 8, gen_group, sprev0)


@jax.jit
def kernel(initial_spikes, input_frames, stacked_flat_spat_filters,
           stacked_feedback_filters, stacked_coupling_filters, stacked_bias,
           coupled_sel, forward_sel, forward_weights):
    frames_p = jnp.pad(input_frames, ((0, M_PAD - N_FRAMES), (0, 0)))

    stim_t = pl.pallas_call(
        _stim_body,
        grid=(N_PIX_CHUNKS,),
        in_specs=[
            pl.BlockSpec((M_PAD, PIX_CHUNK), lambda k: (0, k)),
            pl.BlockSpec((N_CELLS, PIX_CHUNK), lambda k: (0, k)),
            pl.BlockSpec((N_BINS, 2), lambda k: (0, 0)),
            pl.BlockSpec((N_BINS, 2), lambda k: (0, 0)),
            pl.BlockSpec((1, N_CELLS), lambda k: (0, 0)),
        ],
        out_specs=pl.BlockSpec((N_BINS, N_CELLS), lambda k: (0, 0)),
        out_shape=jax.ShapeDtypeStruct((N_BINS, N_CELLS), jnp.float32),
        scratch_shapes=[pltpu.VMEM((M_PAD, N_CELLS), jnp.float32)],
    )(frames_p, stacked_flat_spat_filters,
      forward_sel.astype(jnp.int32), forward_weights,
      stacked_bias.reshape(1, N_CELLS))

    sel_ext = jnp.concatenate(
        [coupled_sel.astype(jnp.int32).T,
         jnp.arange(N_CELLS, dtype=jnp.int32)[None, :]], axis=0)
    e_mat = (jnp.arange(N_CELLS, dtype=jnp.int32)[:, None]
             == sel_ext.reshape(1, N_EDGES)).astype(jnp.bfloat16)
    f_ext = jnp.concatenate(
        [stacked_coupling_filters.transpose(2, 1, 0),
         stacked_feedback_filters.T[:, None, :]], axis=1)
    f_rev = f_ext.reshape(N_TAPS, N_EDGES)[::-1]

    oh = (sel_ext[:, :, None]
          == jnp.arange(N_CELLS, dtype=jnp.int32)[None, None, :])
    frev3 = f_rev[:N_RECENT].reshape(N_RECENT, N_SLOTS, N_CELLS)
    m_stack = jnp.einsum('kcm,jkc->jmc', oh.astype(jnp.float32),
                         frev3).reshape(N_RECENT * N_CELLS,
                                        N_CELLS).astype(jnp.bfloat16)

    stim_p = jnp.pad(stim_t, ((0, 4), (0, 0)))
    init_p = jnp.pad(initial_spikes.T, ((0, 4), (0, 0)))

    out_t = pl.pallas_call(
        _sim_body,
        out_shape=jax.ShapeDtypeStruct((N_BINS + 4, N_CELLS), jnp.float32),
        scratch_shapes=[pltpu.VMEM((ACC_ROWS, N_CELLS), jnp.float32),
                        pltpu.VMEM((N_INIT + 4, N_EDGES), jnp.float32)],
    )(stim_p, init_p, f_rev, e_mat, m_stack)

    return out_t[:N_BINS].T
